# baseline (speedup 1.0000x reference)
import functools

import jax
import jax.numpy as jnp
from jax import lax
from jax.experimental import pallas as pl
from jax.experimental.pallas import tpu as pltpu
from jax.experimental.pallas import tpu_sc as plsc

_LANES = 16
_N_WORKERS = 32
_CHUNK = 6400
_COL_STRIDE = 24


@functools.partial(jax.jit, static_argnames=("nelems",))
def _sc_gather(an_phys, gate_cols, *, nelems):
    total = an_phys.shape[0]
    table_n = gate_cols.shape[0] // nelems
    per_w = total // _N_WORKERS
    n_chunks = per_w // _CHUNK
    groups = _CHUNK // _LANES

    mesh = plsc.VectorSubcoreMesh(core_axis_name="c", subcore_axis_name="s")

    @functools.partial(
        pl.kernel,
        out_type=jax.ShapeDtypeStruct((nelems * total,), jnp.float32),
        mesh=mesh,
        compiler_params=pltpu.CompilerParams(needs_layout_passes=False),
        scratch_types=[
            pltpu.VMEM((_CHUNK,), jnp.int32),
            pltpu.VMEM((_CHUNK,), jnp.int32),
            pltpu.VMEM((nelems * _CHUNK,), jnp.float32),
            pltpu.VMEM((nelems * _CHUNK,), jnp.float32),
            pltpu.VMEM((nelems * table_n,), jnp.float32),
            pltpu.SemaphoreType.DMA,
            pltpu.SemaphoreType.DMA,
            pltpu.SemaphoreType.DMA,
            pltpu.SemaphoreType.DMA,
        ],
    )
    def body(
        an_hbm,
        gate_hbm,
        out_hbm,
        an_v0,
        an_v1,
        out_v0,
        out_v1,
        cols_v,
        sem_i0,
        sem_i1,
        sem_o0,
        sem_o1,
    ):
        wid = lax.axis_index("s") * 2 + lax.axis_index("c")
        base = wid * per_w
        pltpu.sync_copy(gate_hbm, cols_v)

        an_bufs = (an_v0, an_v1)
        out_bufs = (out_v0, out_v1)
        sems_i = (sem_i0, sem_i1)
        sems_o = (sem_o0, sem_o1)

        def start_in(ci, p):
            return pltpu.async_copy(
                an_hbm.at[pl.ds(base + ci * _CHUNK, _CHUNK)],
                an_bufs[p],
                sems_i[p],
            )

        h_in = [start_in(0, 0), None]
        h_out = [None, None]

        for ci in range(n_chunks):
            p = ci % 2
            if ci + 1 < n_chunks:
                h_in[1 - p] = start_in(ci + 1, 1 - p)
            h_in[p].wait()
            if h_out[p] is not None:
                for h in h_out[p]:
                    h.wait()
            an_v = an_bufs[p]
            out_v = out_bufs[p]

            @plsc.parallel_loop(0, groups, 1, unroll=16)
            def group(t):
                an16 = an_v[pl.ds(t * _LANES, _LANES)]
                for k in range(nelems):
                    val = plsc.load_gather(
                        cols_v.at[pl.ds(k * table_n, _LANES + 2)], [an16]
                    )
                    out_v[pl.ds(k * _CHUNK + t * _LANES, _LANES)] = val

            h_out[p] = [
                pltpu.async_copy(
                    out_v.at[pl.ds(k * _CHUNK, _CHUNK)],
                    out_hbm.at[pl.ds(k * total + base + ci * _CHUNK, _CHUNK)],
                    sems_o[p],
                )
                for k in range(nelems)
            ]

        for hs in h_out:
            if hs is not None:
                for h in hs:
                    h.wait()

    return body(an_phys, gate_cols)


def kernel(atomic_numbers, gate_weight):
    b, a = atomic_numbers.shape
    nelems = gate_weight.shape[1]
    total = b * a
    tb, bc = b // 128, 128
    ta, ar = a // 8, 8
    an_phys = (
        atomic_numbers.astype(jnp.int32)
        .reshape(tb, bc, ta, ar)
        .transpose(2, 0, 3, 1)
        .reshape(total)
    )
    gate_cols = jnp.pad(
        gate_weight.astype(jnp.float32).T, ((0, 0), (0, _COL_STRIDE - 18))
    ).reshape(-1)
    out_planes = _sc_gather(an_phys, gate_cols, nelems=nelems)
    return (
        out_planes.reshape(nelems, ta, tb, ar, bc)
        .transpose(2, 4, 1, 3, 0)
        .reshape(b, a, nelems)
    )


# speedup vs baseline: 1.0588x; 1.0588x over previous
import functools

import jax
import jax.numpy as jnp
from jax import lax
from jax.experimental import pallas as pl
from jax.experimental.pallas import tpu as pltpu
from jax.experimental.pallas import tpu_sc as plsc

_LANES = 16
_N_WORKERS = 32
_CHUNK = 6400
_COL_STRIDE = 24


@functools.partial(jax.jit, static_argnames=("nelems",))
def _sc_gather(an_phys, gate_cols, *, nelems):
    total = an_phys.shape[0]
    table_n = gate_cols.shape[0] // nelems
    per_w = total // _N_WORKERS
    n_chunks = per_w // _CHUNK
    groups = _CHUNK // _LANES

    mesh = plsc.VectorSubcoreMesh(core_axis_name="c", subcore_axis_name="s")

    @functools.partial(
        pl.kernel,
        out_type=jax.ShapeDtypeStruct((nelems * total,), jnp.float32),
        mesh=mesh,
        compiler_params=pltpu.CompilerParams(needs_layout_passes=False),
        scratch_types=[
            pltpu.VMEM((_CHUNK,), jnp.int32),
            pltpu.VMEM((_CHUNK,), jnp.int32),
            pltpu.VMEM((nelems * _CHUNK,), jnp.float32),
            pltpu.VMEM((nelems * _CHUNK,), jnp.float32),
            pltpu.VMEM((nelems * table_n,), jnp.float32),
            pltpu.SemaphoreType.DMA,
            pltpu.SemaphoreType.DMA,
            pltpu.SemaphoreType.DMA,
            pltpu.SemaphoreType.DMA,
        ],
    )
    def body(
        an_hbm,
        gate_hbm,
        out_hbm,
        an_v0,
        an_v1,
        out_v0,
        out_v1,
        cols_v,
        sem_i0,
        sem_i1,
        sem_o0,
        sem_o1,
    ):
        wid = lax.axis_index("s") * 2 + lax.axis_index("c")
        base = wid * per_w
        pltpu.sync_copy(gate_hbm, cols_v)

        an_bufs = (an_v0, an_v1)
        out_bufs = (out_v0, out_v1)
        sems_i = (sem_i0, sem_i1)
        sems_o = (sem_o0, sem_o1)

        def start_in(ci, p):
            return pltpu.async_copy(
                an_hbm.at[pl.ds(base + ci * _CHUNK, _CHUNK)],
                an_bufs[p],
                sems_i[p],
            )

        h_in = [start_in(0, 0), None]
        h_out = [None, None]

        for ci in range(n_chunks):
            p = ci % 2
            if ci + 1 < n_chunks:
                h_in[1 - p] = start_in(ci + 1, 1 - p)
            h_in[p].wait()
            if h_out[p] is not None:
                for h in h_out[p]:
                    h.wait()
            an_v = an_bufs[p]
            out_v = out_bufs[p]

            @plsc.parallel_loop(0, groups, 1, unroll=4)
            def group(t):
                an16 = an_v[pl.ds(t * _LANES, _LANES)]
                for k in range(nelems):
                    val = plsc.load_gather(
                        cols_v.at[pl.ds(k * table_n, _LANES + 2)], [an16]
                    )
                    out_v[pl.ds(k * _CHUNK + t * _LANES, _LANES)] = val

            h_out[p] = [
                pltpu.async_copy(
                    out_v.at[pl.ds(k * _CHUNK, _CHUNK)],
                    out_hbm.at[pl.ds(k * total + base + ci * _CHUNK, _CHUNK)],
                    sems_o[p],
                )
                for k in range(nelems)
            ]

        for hs in h_out:
            if hs is not None:
                for h in hs:
                    h.wait()

    return body(an_phys, gate_cols)


def kernel(atomic_numbers, gate_weight):
    b, a = atomic_numbers.shape
    nelems = gate_weight.shape[1]
    total = b * a
    tb, bc = b // 128, 128
    ta, ar = a // 8, 8
    an_phys = (
        atomic_numbers.astype(jnp.int32)
        .reshape(tb, bc, ta, ar)
        .transpose(2, 0, 3, 1)
        .reshape(total)
    )
    gate_cols = jnp.pad(
        gate_weight.astype(jnp.float32).T, ((0, 0), (0, _COL_STRIDE - 18))
    ).reshape(-1)
    out_planes = _sc_gather(an_phys, gate_cols, nelems=nelems)
    return (
        out_planes.reshape(nelems, ta, tb, ar, bc)
        .transpose(2, 4, 1, 3, 0)
        .reshape(b, a, nelems)
    )


# speedup vs baseline: 1.0670x; 1.0077x over previous
import functools

import jax
import jax.numpy as jnp
from jax import lax
from jax.experimental import pallas as pl
from jax.experimental.pallas import tpu as pltpu
from jax.experimental.pallas import tpu_sc as plsc

_LANES = 16
_N_WORKERS = 32
_CHUNK = 6400
_COL_STRIDE = 24


@functools.partial(jax.jit, static_argnames=("nelems",))
def _sc_gather(an_phys, gate_cols, *, nelems):
    total = an_phys.shape[0]
    table_n = gate_cols.shape[0] // nelems
    per_w = total // _N_WORKERS
    n_chunks = per_w // _CHUNK
    groups = _CHUNK // _LANES

    mesh = plsc.VectorSubcoreMesh(core_axis_name="c", subcore_axis_name="s")

    @functools.partial(
        pl.kernel,
        out_type=jax.ShapeDtypeStruct((nelems * total,), jnp.float32),
        mesh=mesh,
        compiler_params=pltpu.CompilerParams(needs_layout_passes=False),
        scratch_types=[
            pltpu.VMEM((_CHUNK,), jnp.int32),
            pltpu.VMEM((_CHUNK,), jnp.int32),
            pltpu.VMEM((nelems * _CHUNK,), jnp.float32),
            pltpu.VMEM((nelems * _CHUNK,), jnp.float32),
            pltpu.VMEM((nelems * table_n,), jnp.float32),
            pltpu.SemaphoreType.DMA,
            pltpu.SemaphoreType.DMA,
            pltpu.SemaphoreType.DMA,
            pltpu.SemaphoreType.DMA,
        ],
    )
    def body(
        an_hbm,
        gate_hbm,
        out_hbm,
        an_v0,
        an_v1,
        out_v0,
        out_v1,
        cols_v,
        sem_i0,
        sem_i1,
        sem_o0,
        sem_o1,
    ):
        wid = lax.axis_index("s") * 2 + lax.axis_index("c")
        base = wid * per_w
        pltpu.sync_copy(gate_hbm, cols_v)

        an_bufs = (an_v0, an_v1)
        out_bufs = (out_v0, out_v1)
        sems_i = (sem_i0, sem_i1)
        sems_o = (sem_o0, sem_o1)

        def start_in(ci, p):
            return pltpu.async_copy(
                an_hbm.at[pl.ds(base + ci * _CHUNK, _CHUNK)],
                an_bufs[p],
                sems_i[p],
            )

        h_in = [start_in(0, 0), None]
        h_out = [None, None]

        for ci in range(n_chunks):
            p = ci % 2
            if ci + 1 < n_chunks:
                h_in[1 - p] = start_in(ci + 1, 1 - p)
            h_in[p].wait()
            if h_out[p] is not None:
                for h in h_out[p]:
                    h.wait()
            an_v = an_bufs[p]
            out_v = out_bufs[p]

            @plsc.parallel_loop(0, groups, 1, unroll=2)
            def group(t):
                an16 = an_v[pl.ds(t * _LANES, _LANES)]
                for k in range(nelems):
                    val = plsc.load_gather(
                        cols_v.at[pl.ds(k * table_n, _LANES + 2)], [an16]
                    )
                    out_v[pl.ds(k * _CHUNK + t * _LANES, _LANES)] = val

            h_out[p] = [
                pltpu.async_copy(
                    out_v.at[pl.ds(k * _CHUNK, _CHUNK)],
                    out_hbm.at[pl.ds(k * total + base + ci * _CHUNK, _CHUNK)],
                    sems_o[p],
                )
                for k in range(nelems)
            ]

        for hs in h_out:
            if hs is not None:
                for h in hs:
                    h.wait()

    return body(an_phys, gate_cols)


def kernel(atomic_numbers, gate_weight):
    b, a = atomic_numbers.shape
    nelems = gate_weight.shape[1]
    total = b * a
    tb, bc = b // 128, 128
    ta, ar = a // 8, 8
    an_phys = (
        atomic_numbers.astype(jnp.int32)
        .reshape(tb, bc, ta, ar)
        .transpose(2, 0, 3, 1)
        .reshape(total)
    )
    gate_cols = jnp.pad(
        gate_weight.astype(jnp.float32).T, ((0, 0), (0, _COL_STRIDE - 18))
    ).reshape(-1)
    out_planes = _sc_gather(an_phys, gate_cols, nelems=nelems)
    return (
        out_planes.reshape(nelems, ta, tb, ar, bc)
        .transpose(2, 4, 1, 3, 0)
        .reshape(b, a, nelems)
    )


# speedup vs baseline: 1.0773x; 1.0097x over previous
import functools

import jax
import jax.numpy as jnp
from jax import lax
from jax.experimental import pallas as pl
from jax.experimental.pallas import tpu as pltpu
from jax.experimental.pallas import tpu_sc as plsc

_LANES = 16
_N_WORKERS = 32
_CHUNK = 6400
_COL_STRIDE = 24


@functools.partial(jax.jit, static_argnames=("nelems",))
def _sc_gather(an_phys, gate_cols, *, nelems):
    total = an_phys.shape[0]
    table_n = gate_cols.shape[0] // nelems
    per_w = total // _N_WORKERS
    n_chunks = per_w // _CHUNK
    groups = _CHUNK // _LANES

    mesh = plsc.VectorSubcoreMesh(core_axis_name="c", subcore_axis_name="s")

    @functools.partial(
        pl.kernel,
        out_type=jax.ShapeDtypeStruct((nelems * total,), jnp.float32),
        mesh=mesh,
        compiler_params=pltpu.CompilerParams(needs_layout_passes=False),
        scratch_types=[
            pltpu.VMEM((_CHUNK,), jnp.int32),
            pltpu.VMEM((_CHUNK,), jnp.int32),
            pltpu.VMEM((nelems * _CHUNK,), jnp.float32),
            pltpu.VMEM((nelems * _CHUNK,), jnp.float32),
            pltpu.VMEM((nelems * table_n,), jnp.float32),
            pltpu.SemaphoreType.DMA,
            pltpu.SemaphoreType.DMA,
            pltpu.SemaphoreType.DMA,
            pltpu.SemaphoreType.DMA,
        ],
    )
    def body(
        an_hbm,
        gate_hbm,
        out_hbm,
        an_v0,
        an_v1,
        out_v0,
        out_v1,
        cols_v,
        sem_i0,
        sem_i1,
        sem_o0,
        sem_o1,
    ):
        wid = lax.axis_index("s") * 2 + lax.axis_index("c")
        base = wid * per_w
        pltpu.sync_copy(gate_hbm, cols_v)

        an_bufs = (an_v0, an_v1)
        out_bufs = (out_v0, out_v1)
        sems_i = (sem_i0, sem_i1)
        sems_o = (sem_o0, sem_o1)

        def start_in(ci, p):
            return pltpu.async_copy(
                an_hbm.at[pl.ds(base + ci * _CHUNK, _CHUNK)],
                an_bufs[p],
                sems_i[p],
            )

        h_in = [start_in(0, 0), None]
        h_out = [None, None]

        for ci in range(n_chunks):
            p = ci % 2
            if ci + 1 < n_chunks:
                h_in[1 - p] = start_in(ci + 1, 1 - p)
            h_in[p].wait()
            if h_out[p] is not None:
                for h in h_out[p]:
                    h.wait()
            an_v = an_bufs[p]
            out_v = out_bufs[p]

            @plsc.parallel_loop(0, groups, 1, unroll=1)
            def group(t):
                an16 = an_v[pl.ds(t * _LANES, _LANES)]
                for k in range(nelems):
                    val = plsc.load_gather(
                        cols_v.at[pl.ds(k * table_n, _LANES + 2)], [an16]
                    )
                    out_v[pl.ds(k * _CHUNK + t * _LANES, _LANES)] = val

            h_out[p] = [
                pltpu.async_copy(
                    out_v.at[pl.ds(k * _CHUNK, _CHUNK)],
                    out_hbm.at[pl.ds(k * total + base + ci * _CHUNK, _CHUNK)],
                    sems_o[p],
                )
                for k in range(nelems)
            ]

        for hs in h_out:
            if hs is not None:
                for h in hs:
                    h.wait()

    return body(an_phys, gate_cols)


def kernel(atomic_numbers, gate_weight):
    b, a = atomic_numbers.shape
    nelems = gate_weight.shape[1]
    total = b * a
    tb, bc = b // 128, 128
    ta, ar = a // 8, 8
    an_phys = (
        atomic_numbers.astype(jnp.int32)
        .reshape(tb, bc, ta, ar)
        .transpose(2, 0, 3, 1)
        .reshape(total)
    )
    gate_cols = jnp.pad(
        gate_weight.astype(jnp.float32).T, ((0, 0), (0, _COL_STRIDE - 18))
    ).reshape(-1)
    out_planes = _sc_gather(an_phys, gate_cols, nelems=nelems)
    return (
        out_planes.reshape(nelems, ta, tb, ar, bc)
        .transpose(2, 4, 1, 3, 0)
        .reshape(b, a, nelems)
    )
